# Initial kernel scaffold; baseline (speedup 1.0000x reference)
import jax
import jax.numpy as jnp
from jax.experimental import pallas as pl


def kernel(x, coordinates, nodal_values, connectivity, cell_id):
    raise NotImplementedError("write your pallas kernel here")


# speedup vs baseline: 9.4049x; 9.4049x over previous
import functools

import jax
import jax.numpy as jnp
from jax import lax
from jax.experimental import pallas as pl
from jax.experimental.pallas import tpu as pltpu
from jax.experimental.pallas import tpu_sc as plsc

NC = 2
NS = 16
NW = NC * NS
L = 16

EC = 1568
PC = 1568


def _mesh():
    return plsc.VectorSubcoreMesh(
        core_axis_name="c", subcore_axis_name="s",
        num_cores=NC, num_subcores=NS)


def _build_stage_a(e_pad):
    n_chunks = e_pad // (NW * EC)
    G = EC // L

    @functools.partial(
        pl.kernel,
        out_type=jax.ShapeDtypeStruct((e_pad * 8,), jnp.float32),
        mesh=_mesh(),
        scratch_types=[
            pltpu.VMEM((EC * 3,), jnp.int32),
            pltpu.VMEM((EC,), jnp.int32),
            pltpu.VMEM((EC,), jnp.int32),
            pltpu.VMEM((EC,), jnp.int32),
            pltpu.VMEM((EC, 4), jnp.float32),
            pltpu.VMEM((EC, 4), jnp.float32),
            pltpu.VMEM((EC, 4), jnp.float32),
            pltpu.VMEM((EC * 8,), jnp.float32),
            pltpu.SemaphoreType.DMA,
        ],
        compiler_params=pltpu.CompilerParams(needs_layout_passes=False, use_tc_tiling_on_sc=False),
    )
    def stage_a(conn_hbm, node_hbm, m_hbm,
                conn_v, idx0, idx1, idx2, r0, r1, r2, m_v, sem):
        wid = lax.axis_index("s") * NC + lax.axis_index("c")
        iota = lax.iota(jnp.int32, L)

        for ch in range(n_chunks):
            ebase = (wid * n_chunks + ch) * EC
            pltpu.sync_copy(conn_hbm.at[pl.ds(ebase * 3, EC * 3)], conn_v)

            @pl.loop(0, G)
            def _extract(g):
                lane3 = iota * 3 + g * (L * 3)
                s = pl.ds(g * L, L)
                idx0[s] = plsc.load_gather(conn_v, [lane3]) - 1
                idx1[s] = plsc.load_gather(conn_v, [lane3 + 1]) - 1
                idx2[s] = plsc.load_gather(conn_v, [lane3 + 2]) - 1

            c0 = pltpu.async_copy(node_hbm.at[idx0], r0, sem)
            c1 = pltpu.async_copy(node_hbm.at[idx1], r1, sem)
            c2 = pltpu.async_copy(node_hbm.at[idx2], r2, sem)
            c0.wait()
            c1.wait()
            c2.wait()

            @pl.loop(0, G)
            def _compute(g):
                lane = iota + g * L
                z = jnp.zeros((L,), jnp.int32)

                def fld(r, c):
                    return plsc.load_gather(r, [lane, z + c])

                x1 = fld(r0, 0); y1 = fld(r0, 1); u1 = fld(r0, 2); w1 = fld(r0, 3)
                x2 = fld(r1, 0); y2 = fld(r1, 1); u2 = fld(r1, 2); w2 = fld(r1, 3)
                x3 = fld(r2, 0); y3 = fld(r2, 1); u3 = fld(r2, 2); w3 = fld(r2, 3)

                d1 = x1 * (y3 - y2) + x2 * (y1 - y3) + x3 * (y2 - y1)
                d2 = -x1 * y2 + x1 * y3 + x2 * y1 - x2 * y3 - x3 * y1 + x3 * y2
                d3 = x1 * (y2 - y3) + x2 * (y3 - y1) + x3 * (y1 - y2)
                m00 = (y3 - y2) / d1
                m10 = (x2 - x3) / d2
                m20 = (x3 * y2 - x2 * y3) / d2
                m01 = (y1 - y3) / d2
                m11 = (x1 - x3) / d3
                m21 = (x3 * y1 - x1 * y3) / d3
                m02 = (y1 - y2) / d3
                m12 = (x1 - x2) / d2
                m22 = (x2 * y1 - x1 * y2) / d2

                a0 = m00 * u1 + m01 * u2 + m02 * u3
                b0 = m10 * u1 + m11 * u2 + m12 * u3
                c0_ = m20 * u1 + m21 * u2 + m22 * u3
                a1 = m00 * w1 + m01 * w2 + m02 * w3
                b1 = m10 * w1 + m11 * w2 + m12 * w3
                c1_ = m20 * w1 + m21 * w2 + m22 * w3

                lane8 = lane * 8
                plsc.store_scatter(m_v, [lane8], a0)
                plsc.store_scatter(m_v, [lane8 + 1], b0)
                plsc.store_scatter(m_v, [lane8 + 2], c0_)
                plsc.store_scatter(m_v, [lane8 + 3], a1)
                plsc.store_scatter(m_v, [lane8 + 4], b1)
                plsc.store_scatter(m_v, [lane8 + 5], c1_)

            pltpu.sync_copy(m_v, m_hbm.at[pl.ds(ebase * 8, EC * 8)])

    return stage_a


def _build_stage_b(p_pad, e_pad):
    n_chunks = p_pad // (NW * PC)
    G = PC // L

    @functools.partial(
        pl.kernel,
        out_type=jax.ShapeDtypeStruct((2, p_pad), jnp.float32),
        mesh=_mesh(),
        scratch_types=[
            pltpu.VMEM((PC,), jnp.int32),
            pltpu.VMEM((PC * 2,), jnp.float32),
            pltpu.VMEM((PC, 8), jnp.float32),
            pltpu.VMEM((PC,), jnp.float32),
            pltpu.VMEM((PC,), jnp.float32),
            pltpu.SemaphoreType.DMA,
        ],
        compiler_params=pltpu.CompilerParams(needs_layout_passes=False, use_tc_tiling_on_sc=False),
    )
    def stage_b(x_hbm, cid_hbm, m_hbm, out_hbm,
                cid_v, x_v, rows_v, o0, o1, sem):
        wid = lax.axis_index("s") * NC + lax.axis_index("c")
        iota = lax.iota(jnp.int32, L)

        for ch in range(n_chunks):
            pbase = (wid * n_chunks + ch) * PC
            pltpu.sync_copy(cid_hbm.at[pl.ds(pbase, PC)], cid_v)
            pltpu.sync_copy(x_hbm.at[pl.ds(pbase * 2, PC * 2)], x_v)
            pltpu.async_copy(m_hbm.at[cid_v], rows_v, sem).wait()

            @pl.loop(0, G)
            def _cmp(g):
                lane = iota + g * L
                z = jnp.zeros((L,), jnp.int32)

                def fld(c):
                    return plsc.load_gather(rows_v, [lane, z + c])

                lane2 = lane * 2
                xx = plsc.load_gather(x_v, [lane2])
                yy = plsc.load_gather(x_v, [lane2 + 1])
                a0 = fld(0); b0 = fld(1); c0 = fld(2)
                a1 = fld(3); b1 = fld(4); c1 = fld(5)
                s = pl.ds(g * L, L)
                o0[s] = xx * a0 + yy * b0 + c0
                o1[s] = xx * a1 + yy * b1 + c1

            pltpu.sync_copy(o0, out_hbm.at[0, pl.ds(pbase, PC)])
            pltpu.sync_copy(o1, out_hbm.at[1, pl.ds(pbase, PC)])

    return stage_b


def _round_up(v, m):
    return (v + m - 1) // m * m


def kernel(x, coordinates, nodal_values, connectivity, cell_id):
    p = x.shape[0]
    e = connectivity.shape[0]

    e_pad = _round_up(e, NW * EC)
    p_pad = _round_up(p, NW * PC)

    node_tab = jnp.concatenate(
        [coordinates, nodal_values.T.astype(jnp.float32)], axis=1)

    conn_flat = jnp.pad(connectivity.astype(jnp.int32),
                        ((0, e_pad - e), (0, 0)), constant_values=1).reshape(-1)

    m_flat = _build_stage_a(e_pad)(conn_flat, node_tab)
    m_tab = m_flat.reshape(e_pad, 8)

    x_flat = jnp.pad(x, ((0, p_pad - p), (0, 0))).reshape(-1)
    cid_pad = jnp.pad(cell_id.astype(jnp.int32), (0, p_pad - p))

    out = _build_stage_b(p_pad, e_pad)(x_flat, cid_pad, m_tab)
    return out[:, :p]
